# baseline (speedup 1.0000x reference)
import functools

import jax
import jax.numpy as jnp
import numpy as np
from jax import lax
from jax.experimental import pallas as pl
from jax.experimental.pallas import tpu as pltpu
from jax.experimental.pallas import tpu_sc as plsc

N = 50000
K = 16
T = 512
NPARAM = 8
SMOOTH = 0.1
PERIODS = (0.25, 0.5, 1.0, 2.0)

NH = N // 2
HPAD = 25600
CHUNK = 800
NPAD = 51200
NCHUNKS = NPAD // CHUNK
QCHUNKS = NCHUNKS // 4


def _sc_smooth_body(params_hbm, idx_hbm, w_hbm, out_hbm,
                    table_v, idx_v, w_v, out_v0, out_v1,
                    sem_idx, sem_w, sem_out):
    out_bufs = (out_v0, out_v1)
    cid = lax.axis_index("c")
    sid = lax.axis_index("s")
    p = sid % NPARAM
    qq = sid // NPARAM
    q = cid * 2 + qq

    p_off = pl.multiple_of(p * NPAD, NPAD)
    pltpu.sync_copy(params_hbm.at[pl.ds(p_off, NPAD)], table_v)

    def in_copies(blk):
        buf = blk % 2
        g = q * QCHUNKS + blk
        return (
            pltpu.make_async_copy(idx_hbm.at[g], idx_v.at[buf],
                                  sem_idx.at[buf]),
            pltpu.make_async_copy(w_hbm.at[g], w_v.at[buf],
                                  sem_w.at[buf]),
        )

    def out_copy(blk):
        buf = blk % 2
        o_off = pl.multiple_of(p_off + (q * QCHUNKS + blk) * CHUNK, CHUNK)
        return pltpu.make_async_copy(out_bufs[buf],
                                     out_hbm.at[pl.ds(o_off, CHUNK)],
                                     sem_out.at[buf])

    for c in in_copies(0):
        c.start()
    for blk in range(QCHUNKS):
        buf = blk % 2
        if blk + 1 < QCHUNKS:
            for c in in_copies(blk + 1):
                c.start()
        for c in in_copies(blk):
            c.wait()
        if blk >= 2:
            out_copy(blk - 2).wait()
        base = (q * QCHUNKS + blk) * CHUNK - jnp.where(q >= 2, HPAD - NH, 0)

        def body(gi, _, buf=buf, base=base):
            col = gi * 16
            iv = idx_v[buf, 0, pl.ds(col, 16)]
            acc = w_v[buf, 0, pl.ds(col, 16)] * plsc.load_gather(table_v, [iv])
            for k in range(1, K):
                iv = idx_v[buf, k, pl.ds(col, 16)]
                acc = acc + w_v[buf, k, pl.ds(col, 16)] * plsc.load_gather(table_v, [iv])
            center = table_v[pl.ds(base + col, 16)]
            out_bufs[buf][pl.ds(col, 16)] = (1.0 - SMOOTH) * center + SMOOTH * acc
            return 0

        lax.fori_loop(0, CHUNK // 16, body, 0)
        out_copy(blk).start()
    for blk in (QCHUNKS - 2, QCHUNKS - 1):
        out_copy(blk).wait()


_sc_cache = {}


def _get_sc_smooth():
    if "k" not in _sc_cache:
        _sc_cache["k"] = pl.kernel(
            _sc_smooth_body,
            out_type=jax.ShapeDtypeStruct((NPARAM * NPAD,), jnp.float32),
            mesh=plsc.VectorSubcoreMesh(core_axis_name="c",
                                        subcore_axis_name="s"),
            scratch_types=[
                pltpu.VMEM((NPAD,), jnp.float32),
                pltpu.VMEM((2, K, CHUNK), jnp.int32),
                pltpu.VMEM((2, K, CHUNK), jnp.float32),
                pltpu.VMEM((CHUNK,), jnp.float32),
                pltpu.VMEM((CHUNK,), jnp.float32),
                pltpu.SemaphoreType.DMA((2,)),
                pltpu.SemaphoreType.DMA((2,)),
                pltpu.SemaphoreType.DMA((2,)),
            ],
            compiler_params=pltpu.CompilerParams(needs_layout_passes=False),
        )
    return _sc_cache["k"]


_HALF_PI = np.float32(np.pi / 2)
_QUARTER_PI = np.float32(np.pi / 4)
_TWO_OVER_PI = np.float32(2.0 / np.pi)
_INV_SQRT2 = np.float32(1.0 / np.sqrt(2.0))


def _fast_cos(x):
    u = x * _TWO_OVER_PI
    kf = jnp.floor(u)
    t = (u - kf) * _HALF_PI - _QUARTER_PI
    t2 = t * t
    st = t * (1.0 + t2 * (np.float32(-1.0 / 6.0) + t2 * (
        np.float32(1.0 / 120.0) + t2 * np.float32(-1.0 / 5040.0))))
    ct = 1.0 + t2 * (np.float32(-0.5) + t2 * (
        np.float32(1.0 / 24.0) + t2 * np.float32(-1.0 / 720.0)))
    s = (ct + st) * _INV_SQRT2
    c = (ct - st) * _INV_SQRT2
    ki = kf.astype(jnp.int32)
    mag = jnp.where((ki & 1) == 1, s, c)
    return jnp.where(((ki + 1) & 2) == 2, -mag, mag)


def _bf16_split(x):
    hi = x.astype(jnp.bfloat16).astype(jnp.float32)
    return hi, x - hi


def _coeff_body(time_ref, off_ref, trend_ref, sm_ref, basis_ref, co_ref):
    t = time_ref[...]
    args = []
    for period in PERIODS:
        w = np.float32(2.0 * np.pi / period)
        args.append(w * t - _HALF_PI)
    for period in PERIODS:
        w = np.float32(2.0 * np.pi / period)
        args.append(w * t)
    sincos_t = _fast_cos(jnp.concatenate(args, axis=0))
    t_hi, t_lo = _bf16_split(t)
    ones = jnp.ones_like(t)
    basis_ref[...] = jnp.concatenate(
        [sincos_t, ones, ones, t_hi, t_lo, t_hi,
         jnp.zeros((3, T), jnp.float32)], axis=0)

    amp = sm_ref[0:4, :]
    ph = sm_ref[4:8, :]
    sc = _fast_cos(jnp.concatenate([ph - _HALF_PI, ph], axis=0))
    off_hi, off_lo = _bf16_split(off_ref[...])
    tr_hi, tr_lo = _bf16_split(trend_ref[...])
    co = jnp.concatenate(
        [amp * sc[4:8, :],
         amp * sc[0:4, :],
         off_hi, off_lo, tr_hi, tr_hi, tr_lo,
         jnp.zeros((3, HPAD), jnp.float32)], axis=0)
    co_ref[...] = jnp.transpose(co)


def _coeffs(time2, off2, trend2, sm):
    return pl.pallas_call(
        _coeff_body,
        out_shape=(
            jax.ShapeDtypeStruct((16, T), jnp.float32),
            jax.ShapeDtypeStruct((HPAD, 16), jnp.float32),
        ),
    )(time2, off2, trend2, sm)


_ROWS = 1000


def _dense_body(basis_ref, co_ref, out_ref):
    out_ref[...] = jax.lax.dot_general(
        co_ref[...], basis_ref[...],
        dimension_numbers=(((1,), (0,)), ((), ())),
        preferred_element_type=jnp.float32)


def _dense_body_carry(basis_ref, co_ref, _, out_ref):
    _dense_body(basis_ref, co_ref, out_ref)


def _dense_half(half, basis, co, carry):
    nblk = NH // _ROWS
    in_specs = [
        pl.BlockSpec((16, T), lambda i: (0, 0)),
        pl.BlockSpec((_ROWS, 16), lambda i: (i, 0)),
    ]
    args = [basis, co]
    body = _dense_body
    aliases = {}
    if carry is not None:
        in_specs.append(pl.BlockSpec(memory_space=pl.ANY))
        args.append(carry)
        body = _dense_body_carry
        aliases = {2: 0}
    return pl.pallas_call(
        body,
        grid=(nblk,),
        in_specs=in_specs,
        out_specs=pl.BlockSpec((_ROWS, T),
                               lambda i, h=half: (h * nblk + i, 0)),
        out_shape=jax.ShapeDtypeStruct((N, T), jnp.float32),
        input_output_aliases=aliases,
    )(*args)


def kernel(time_vector, constant_offset, seasonal_amplitudes, seasonal_phases,
           linear_trend, neighbor_indices, neighbor_weights):
    params8 = jnp.concatenate([seasonal_amplitudes.T, seasonal_phases.T], axis=0)
    params8 = jnp.pad(params8, ((0, 0), (0, NPAD - N))).reshape(NPARAM * NPAD)
    idxT = neighbor_indices.T.astype(jnp.int32)
    wT = neighbor_weights.T
    idx_hp = jnp.concatenate(
        [jnp.pad(idxT[:, h * NH:(h + 1) * NH], ((0, 0), (0, HPAD - NH)))
         for h in range(2)], axis=1)
    w_hp = jnp.concatenate(
        [jnp.pad(wT[:, h * NH:(h + 1) * NH], ((0, 0), (0, HPAD - NH)))
         for h in range(2)], axis=1)
    idx_blocks = idx_hp.reshape(K, NCHUNKS, CHUNK).transpose(1, 0, 2)
    w_blocks = w_hp.reshape(K, NCHUNKS, CHUNK).transpose(1, 0, 2)

    smoothed = _get_sc_smooth()(params8, idx_blocks, w_blocks)
    smoothed = smoothed.reshape(NPARAM, NPAD)

    time2 = time_vector[None, :]
    stage2 = []
    for half in range(2):
        sm_h = smoothed[:, half * HPAD:(half + 1) * HPAD]
        off_h = jnp.pad(constant_offset[half * NH:(half + 1) * NH],
                        (0, HPAD - NH))[None, :]
        tr_h = jnp.pad(linear_trend[half * NH:(half + 1) * NH],
                       (0, HPAD - NH))[None, :]
        stage2.append(_coeffs(time2, off_h, tr_h, sm_h))
    carry = None
    for half in range(2):
        basis, co = stage2[half]
        carry = _dense_half(half, basis, co, carry)
    return carry


# speedup vs baseline: 1.1527x; 1.1527x over previous
import functools

import jax
import jax.numpy as jnp
import numpy as np
from jax import lax
from jax.experimental import pallas as pl
from jax.experimental.pallas import tpu as pltpu
from jax.experimental.pallas import tpu_sc as plsc

N = 50000
K = 16
T = 512
NPARAM = 8
SMOOTH = 0.1
PERIODS = (0.25, 0.5, 1.0, 2.0)

NH = N // 2
HPAD = 25600
CHUNK = 800
HCHUNKS = HPAD // CHUNK
QCHUNKS = HCHUNKS // 4
NPAD = 2 * HPAD


def _sc_smooth_body(half_base, params_hbm, idx_hbm, w_hbm, out_hbm,
                    table_v, idx_v, w_v, out_v0, out_v1,
                    sem_idx, sem_w, sem_out):
    out_bufs = (out_v0, out_v1)
    cid = lax.axis_index("c")
    sid = lax.axis_index("s")
    p = sid % NPARAM
    qq = sid // NPARAM
    q = cid * 2 + qq

    p_off = pl.multiple_of(p * NPAD, NPAD)
    pltpu.sync_copy(params_hbm.at[pl.ds(p_off, NPAD)], table_v)

    def in_copies(blk):
        buf = blk % 2
        g = q * QCHUNKS + blk
        return (
            pltpu.make_async_copy(idx_hbm.at[g], idx_v.at[buf],
                                  sem_idx.at[buf]),
            pltpu.make_async_copy(w_hbm.at[g], w_v.at[buf],
                                  sem_w.at[buf]),
        )

    def out_copy(blk):
        buf = blk % 2
        o_off = pl.multiple_of(p * HPAD + (q * QCHUNKS + blk) * CHUNK, CHUNK)
        return pltpu.make_async_copy(out_bufs[buf],
                                     out_hbm.at[pl.ds(o_off, CHUNK)],
                                     sem_out.at[buf])

    for c in in_copies(0):
        c.start()
    for blk in range(QCHUNKS):
        buf = blk % 2
        if blk + 1 < QCHUNKS:
            for c in in_copies(blk + 1):
                c.start()
        for c in in_copies(blk):
            c.wait()
        if blk >= 2:
            out_copy(blk - 2).wait()
        base = half_base + (q * QCHUNKS + blk) * CHUNK

        def body(gi, _, buf=buf, base=base):
            col = gi * 16
            iv = idx_v[buf, 0, pl.ds(col, 16)]
            acc = w_v[buf, 0, pl.ds(col, 16)] * plsc.load_gather(table_v, [iv])
            for k in range(1, K):
                iv = idx_v[buf, k, pl.ds(col, 16)]
                acc = acc + w_v[buf, k, pl.ds(col, 16)] * plsc.load_gather(table_v, [iv])
            center = table_v[pl.ds(base + col, 16)]
            out_bufs[buf][pl.ds(col, 16)] = (1.0 - SMOOTH) * center + SMOOTH * acc
            return 0

        lax.fori_loop(0, CHUNK // 16, body, 0)
        out_copy(blk).start()
    for blk in (QCHUNKS - 2, QCHUNKS - 1):
        out_copy(blk).wait()


_sc_cache = {}


def _get_sc_smooth(half):
    if half not in _sc_cache:
        _sc_cache[half] = pl.kernel(
            functools.partial(_sc_smooth_body, half * NH),
            out_type=jax.ShapeDtypeStruct((NPARAM * HPAD,), jnp.float32),
            mesh=plsc.VectorSubcoreMesh(core_axis_name="c",
                                        subcore_axis_name="s"),
            scratch_types=[
                pltpu.VMEM((NPAD,), jnp.float32),
                pltpu.VMEM((2, K, CHUNK), jnp.int32),
                pltpu.VMEM((2, K, CHUNK), jnp.float32),
                pltpu.VMEM((CHUNK,), jnp.float32),
                pltpu.VMEM((CHUNK,), jnp.float32),
                pltpu.SemaphoreType.DMA((2,)),
                pltpu.SemaphoreType.DMA((2,)),
                pltpu.SemaphoreType.DMA((2,)),
            ],
            compiler_params=pltpu.CompilerParams(needs_layout_passes=False),
        )
    return _sc_cache[half]


_HALF_PI = np.float32(np.pi / 2)
_QUARTER_PI = np.float32(np.pi / 4)
_TWO_OVER_PI = np.float32(2.0 / np.pi)
_INV_SQRT2 = np.float32(1.0 / np.sqrt(2.0))


def _fast_cos(x):
    u = x * _TWO_OVER_PI
    kf = jnp.floor(u)
    t = (u - kf) * _HALF_PI - _QUARTER_PI
    t2 = t * t
    st = t * (1.0 + t2 * (np.float32(-1.0 / 6.0) + t2 * (
        np.float32(1.0 / 120.0) + t2 * np.float32(-1.0 / 5040.0))))
    ct = 1.0 + t2 * (np.float32(-0.5) + t2 * (
        np.float32(1.0 / 24.0) + t2 * np.float32(-1.0 / 720.0)))
    s = (ct + st) * _INV_SQRT2
    c = (ct - st) * _INV_SQRT2
    ki = kf.astype(jnp.int32)
    mag = jnp.where((ki & 1) == 1, s, c)
    return jnp.where(((ki + 1) & 2) == 2, -mag, mag)


def _bf16_split(x):
    hi = x.astype(jnp.bfloat16).astype(jnp.float32)
    return hi, x - hi


def _coeff_body(time_ref, off_ref, trend_ref, sm_ref, basis_ref, co_ref):
    t = time_ref[...]
    args = []
    for period in PERIODS:
        w = np.float32(2.0 * np.pi / period)
        args.append(w * t - _HALF_PI)
    for period in PERIODS:
        w = np.float32(2.0 * np.pi / period)
        args.append(w * t)
    sincos_t = _fast_cos(jnp.concatenate(args, axis=0))
    t_hi, t_lo = _bf16_split(t)
    ones = jnp.ones_like(t)
    basis_ref[...] = jnp.concatenate(
        [sincos_t, ones, ones, t_hi, t_lo, t_hi,
         jnp.zeros((3, T), jnp.float32)], axis=0)

    amp = sm_ref[0:4, :]
    ph = sm_ref[4:8, :]
    sc = _fast_cos(jnp.concatenate([ph - _HALF_PI, ph], axis=0))
    off_hi, off_lo = _bf16_split(off_ref[...])
    tr_hi, tr_lo = _bf16_split(trend_ref[...])
    co = jnp.concatenate(
        [amp * sc[4:8, :],
         amp * sc[0:4, :],
         off_hi, off_lo, tr_hi, tr_hi, tr_lo,
         jnp.zeros((3, HPAD), jnp.float32)], axis=0)
    co_ref[...] = jnp.transpose(co)


def _coeffs(time2, off2, trend2, sm):
    return pl.pallas_call(
        _coeff_body,
        out_shape=(
            jax.ShapeDtypeStruct((16, T), jnp.float32),
            jax.ShapeDtypeStruct((HPAD, 16), jnp.float32),
        ),
    )(time2, off2, trend2, sm)


_ROWS = 1000


def _dense_body(basis_ref, co_ref, out_ref):
    out_ref[...] = jax.lax.dot_general(
        co_ref[...], basis_ref[...],
        dimension_numbers=(((1,), (0,)), ((), ())),
        preferred_element_type=jnp.float32)


def _dense_body_carry(basis_ref, co_ref, _, out_ref):
    _dense_body(basis_ref, co_ref, out_ref)


def _dense_half(half, basis, co, carry):
    nblk = NH // _ROWS
    in_specs = [
        pl.BlockSpec((16, T), lambda i: (0, 0)),
        pl.BlockSpec((_ROWS, 16), lambda i: (i, 0)),
    ]
    args = [basis, co]
    body = _dense_body
    aliases = {}
    if carry is not None:
        in_specs.append(pl.BlockSpec(memory_space=pl.ANY))
        args.append(carry)
        body = _dense_body_carry
        aliases = {2: 0}
    return pl.pallas_call(
        body,
        grid=(nblk,),
        in_specs=in_specs,
        out_specs=pl.BlockSpec((_ROWS, T),
                               lambda i, h=half: (h * nblk + i, 0)),
        out_shape=jax.ShapeDtypeStruct((N, T), jnp.float32),
        input_output_aliases=aliases,
    )(*args)


def kernel(time_vector, constant_offset, seasonal_amplitudes, seasonal_phases,
           linear_trend, neighbor_indices, neighbor_weights):
    params8 = jnp.concatenate([seasonal_amplitudes.T, seasonal_phases.T], axis=0)
    params8 = jnp.pad(params8, ((0, 0), (0, NPAD - N))).reshape(NPARAM * NPAD)
    idxT = neighbor_indices.T.astype(jnp.int32)
    wT = neighbor_weights.T
    halves_idx = []
    halves_w = []
    for half in range(2):
        ih = jnp.pad(idxT[:, half * NH:(half + 1) * NH], ((0, 0), (0, HPAD - NH)))
        wh = jnp.pad(wT[:, half * NH:(half + 1) * NH], ((0, 0), (0, HPAD - NH)))
        halves_idx.append(ih.reshape(K, HCHUNKS, CHUNK).transpose(1, 0, 2))
        halves_w.append(wh.reshape(K, HCHUNKS, CHUNK).transpose(1, 0, 2))

    time2 = time_vector[None, :]
    sms = [_get_sc_smooth(half)(params8, halves_idx[half], halves_w[half])
           for half in range(2)]
    stage2 = []
    for half in range(2):
        sm = sms[half].reshape(NPARAM, HPAD)
        off_h = jnp.pad(constant_offset[half * NH:(half + 1) * NH],
                        (0, HPAD - NH))[None, :]
        tr_h = jnp.pad(linear_trend[half * NH:(half + 1) * NH],
                       (0, HPAD - NH))[None, :]
        stage2.append(_coeffs(time2, off_h, tr_h, sm))
    carry = None
    for half in range(2):
        basis, co = stage2[half]
        carry = _dense_half(half, basis, co, carry)
    return carry


# speedup vs baseline: 1.2805x; 1.1109x over previous
import functools

import jax
import jax.numpy as jnp
import numpy as np
from jax import lax
from jax.experimental import pallas as pl
from jax.experimental.pallas import tpu as pltpu
from jax.experimental.pallas import tpu_sc as plsc

N = 50000
K = 16
T = 512
NPARAM = 8
SMOOTH = 0.1
PERIODS = (0.25, 0.5, 1.0, 2.0)

NH = N // 2
HPAD = 25600
CHUNK = 800
HCHUNKS = HPAD // CHUNK
RCHUNKS = HCHUNKS // 8
NPAD = 2 * HPAD


_MASK_HI = np.int32(-65536)


def _unpack_pair(g32):
    amp = plsc.bitcast(g32 & _MASK_HI, jnp.float32)
    ph = plsc.bitcast(g32 << 16, jnp.float32)
    return amp, ph


def _sc_smooth_body(half_base, params_hbm, idx_hbm, w_hbm, out_hbm,
                    table_v, idx_v, w_v, oa0, oa1, op0, op1,
                    sem_idx, sem_w, sem_oa, sem_op):
    oa = (oa0, oa1)
    op = (op0, op1)
    cid = lax.axis_index("c")
    sid = lax.axis_index("s")
    r = sid % 4
    e = (sid // 4) * 2 + cid

    t_off = pl.multiple_of(r * NPAD, NPAD)
    pltpu.sync_copy(params_hbm.at[pl.ds(t_off, NPAD)], table_v)

    def in_copies(blk):
        buf = blk % 2
        g = e * RCHUNKS + blk
        return (
            pltpu.make_async_copy(idx_hbm.at[g], idx_v.at[buf],
                                  sem_idx.at[buf]),
            pltpu.make_async_copy(w_hbm.at[g], w_v.at[buf],
                                  sem_w.at[buf]),
        )

    def out_copies(blk):
        buf = blk % 2
        c_i = e * RCHUNKS + blk
        o_a = pl.multiple_of(r * HPAD + c_i * CHUNK, CHUNK)
        o_p = pl.multiple_of((r + 4) * HPAD + c_i * CHUNK, CHUNK)
        return (
            pltpu.make_async_copy(oa[buf], out_hbm.at[pl.ds(o_a, CHUNK)],
                                  sem_oa.at[buf]),
            pltpu.make_async_copy(op[buf], out_hbm.at[pl.ds(o_p, CHUNK)],
                                  sem_op.at[buf]),
        )

    for c in in_copies(0):
        c.start()
    for blk in range(RCHUNKS):
        buf = blk % 2
        if blk + 1 < RCHUNKS:
            for c in in_copies(blk + 1):
                c.start()
        for c in in_copies(blk):
            c.wait()
        if blk >= 2:
            for c in out_copies(blk - 2):
                c.wait()
        base = half_base + (e * RCHUNKS + blk) * CHUNK

        def body(gi, _, buf=buf, base=base):
            col = gi * 16
            g32 = plsc.load_gather(table_v, [idx_v[buf, 0, pl.ds(col, 16)]])
            amp, ph = _unpack_pair(g32)
            wv = w_v[buf, 0, pl.ds(col, 16)]
            acc_a = wv * amp
            acc_p = wv * ph
            for k in range(1, K):
                g32 = plsc.load_gather(table_v, [idx_v[buf, k, pl.ds(col, 16)]])
                amp, ph = _unpack_pair(g32)
                wv = w_v[buf, k, pl.ds(col, 16)]
                acc_a = acc_a + wv * amp
                acc_p = acc_p + wv * ph
            ca, cp = _unpack_pair(table_v[pl.ds(base + col, 16)])
            oa[buf][pl.ds(col, 16)] = (1.0 - SMOOTH) * ca + SMOOTH * acc_a
            op[buf][pl.ds(col, 16)] = (1.0 - SMOOTH) * cp + SMOOTH * acc_p
            return 0

        lax.fori_loop(0, CHUNK // 16, body, 0)
        for c in out_copies(blk):
            c.start()
    for blk in (RCHUNKS - 2, RCHUNKS - 1):
        for c in out_copies(blk):
            c.wait()


_sc_cache = {}


def _get_sc_smooth(half):
    if half not in _sc_cache:
        _sc_cache[half] = pl.kernel(
            functools.partial(_sc_smooth_body, half * NH),
            out_type=jax.ShapeDtypeStruct((NPARAM * HPAD,), jnp.float32),
            mesh=plsc.VectorSubcoreMesh(core_axis_name="c",
                                        subcore_axis_name="s"),
            scratch_types=[
                pltpu.VMEM((NPAD,), jnp.int32),
                pltpu.VMEM((2, K, CHUNK), jnp.int32),
                pltpu.VMEM((2, K, CHUNK), jnp.float32),
                pltpu.VMEM((CHUNK,), jnp.float32),
                pltpu.VMEM((CHUNK,), jnp.float32),
                pltpu.VMEM((CHUNK,), jnp.float32),
                pltpu.VMEM((CHUNK,), jnp.float32),
                pltpu.SemaphoreType.DMA((2,)),
                pltpu.SemaphoreType.DMA((2,)),
                pltpu.SemaphoreType.DMA((2,)),
                pltpu.SemaphoreType.DMA((2,)),
            ],
            compiler_params=pltpu.CompilerParams(needs_layout_passes=False),
        )
    return _sc_cache[half]


_HALF_PI = np.float32(np.pi / 2)
_QUARTER_PI = np.float32(np.pi / 4)
_TWO_OVER_PI = np.float32(2.0 / np.pi)
_INV_SQRT2 = np.float32(1.0 / np.sqrt(2.0))


def _fast_cos(x):
    u = x * _TWO_OVER_PI
    kf = jnp.floor(u)
    t = (u - kf) * _HALF_PI - _QUARTER_PI
    t2 = t * t
    st = t * (1.0 + t2 * (np.float32(-1.0 / 6.0) + t2 * (
        np.float32(1.0 / 120.0) + t2 * np.float32(-1.0 / 5040.0))))
    ct = 1.0 + t2 * (np.float32(-0.5) + t2 * (
        np.float32(1.0 / 24.0) + t2 * np.float32(-1.0 / 720.0)))
    s = (ct + st) * _INV_SQRT2
    c = (ct - st) * _INV_SQRT2
    ki = kf.astype(jnp.int32)
    mag = jnp.where((ki & 1) == 1, s, c)
    return jnp.where(((ki + 1) & 2) == 2, -mag, mag)


def _bf16_split(x):
    hi = x.astype(jnp.bfloat16).astype(jnp.float32)
    return hi, x - hi


def _coeff_body(time_ref, off_ref, trend_ref, sm_ref, basis_ref, co_ref):
    t = time_ref[...]
    args = []
    for period in PERIODS:
        w = np.float32(2.0 * np.pi / period)
        args.append(w * t - _HALF_PI)
    for period in PERIODS:
        w = np.float32(2.0 * np.pi / period)
        args.append(w * t)
    sincos_t = _fast_cos(jnp.concatenate(args, axis=0))
    t_hi, t_lo = _bf16_split(t)
    ones = jnp.ones_like(t)
    basis_ref[...] = jnp.concatenate(
        [sincos_t, ones, ones, t_hi, t_lo, t_hi,
         jnp.zeros((3, T), jnp.float32)], axis=0)

    amp = sm_ref[0:4, :]
    ph = sm_ref[4:8, :]
    sc = _fast_cos(jnp.concatenate([ph - _HALF_PI, ph], axis=0))
    off_hi, off_lo = _bf16_split(off_ref[...])
    tr_hi, tr_lo = _bf16_split(trend_ref[...])
    co = jnp.concatenate(
        [amp * sc[4:8, :],
         amp * sc[0:4, :],
         off_hi, off_lo, tr_hi, tr_hi, tr_lo,
         jnp.zeros((3, HPAD), jnp.float32)], axis=0)
    co_ref[...] = jnp.transpose(co)


def _coeffs(time2, off2, trend2, sm):
    return pl.pallas_call(
        _coeff_body,
        out_shape=(
            jax.ShapeDtypeStruct((16, T), jnp.float32),
            jax.ShapeDtypeStruct((HPAD, 16), jnp.float32),
        ),
    )(time2, off2, trend2, sm)


_ROWS = 1000


def _dense_body(basis_ref, co_ref, out_ref):
    out_ref[...] = jax.lax.dot_general(
        co_ref[...], basis_ref[...],
        dimension_numbers=(((1,), (0,)), ((), ())),
        preferred_element_type=jnp.float32)


def _dense_body_carry(basis_ref, co_ref, _, out_ref):
    _dense_body(basis_ref, co_ref, out_ref)


def _dense_half(half, basis, co, carry):
    nblk = NH // _ROWS
    in_specs = [
        pl.BlockSpec((16, T), lambda i: (0, 0)),
        pl.BlockSpec((_ROWS, 16), lambda i: (i, 0)),
    ]
    args = [basis, co]
    body = _dense_body
    aliases = {}
    if carry is not None:
        in_specs.append(pl.BlockSpec(memory_space=pl.ANY))
        args.append(carry)
        body = _dense_body_carry
        aliases = {2: 0}
    return pl.pallas_call(
        body,
        grid=(nblk,),
        in_specs=in_specs,
        out_specs=pl.BlockSpec((_ROWS, T),
                               lambda i, h=half: (h * nblk + i, 0)),
        out_shape=jax.ShapeDtypeStruct((N, T), jnp.float32),
        input_output_aliases=aliases,
    )(*args)


def kernel(time_vector, constant_offset, seasonal_amplitudes, seasonal_phases,
           linear_trend, neighbor_indices, neighbor_weights):
    amp_u = lax.bitcast_convert_type(
        seasonal_amplitudes.T.astype(jnp.bfloat16), jnp.uint16)
    ph_u = lax.bitcast_convert_type(
        seasonal_phases.T.astype(jnp.bfloat16), jnp.uint16)
    packed = (amp_u.astype(jnp.uint32) << 16) | ph_u.astype(jnp.uint32)
    packed = lax.bitcast_convert_type(packed, jnp.int32)
    packed = jnp.pad(packed, ((0, 0), (0, NPAD - N))).reshape(4 * NPAD)
    idxT = neighbor_indices.T.astype(jnp.int32)
    wT = neighbor_weights.T
    halves_idx = []
    halves_w = []
    for half in range(2):
        ih = jnp.pad(idxT[:, half * NH:(half + 1) * NH], ((0, 0), (0, HPAD - NH)))
        wh = jnp.pad(wT[:, half * NH:(half + 1) * NH], ((0, 0), (0, HPAD - NH)))
        halves_idx.append(ih.reshape(K, HCHUNKS, CHUNK).transpose(1, 0, 2))
        halves_w.append(wh.reshape(K, HCHUNKS, CHUNK).transpose(1, 0, 2))

    time2 = time_vector[None, :]
    sms = [_get_sc_smooth(half)(packed, halves_idx[half], halves_w[half])
           for half in range(2)]
    stage2 = []
    for half in range(2):
        sm = sms[half].reshape(NPARAM, HPAD)
        off_h = jnp.pad(constant_offset[half * NH:(half + 1) * NH],
                        (0, HPAD - NH))[None, :]
        tr_h = jnp.pad(linear_trend[half * NH:(half + 1) * NH],
                       (0, HPAD - NH))[None, :]
        stage2.append(_coeffs(time2, off_h, tr_h, sm))
    carry = None
    for half in range(2):
        basis, co = stage2[half]
        carry = _dense_half(half, basis, co, carry)
    return carry


# speedup vs baseline: 1.4730x; 1.1503x over previous
import functools

import jax
import jax.numpy as jnp
import numpy as np
from jax import lax
from jax.experimental import pallas as pl
from jax.experimental.pallas import tpu as pltpu
from jax.experimental.pallas import tpu_sc as plsc

N = 50000
K = 16
T = 512
NPARAM = 8
SMOOTH = 0.1
PERIODS = (0.25, 0.5, 1.0, 2.0)

NH = N // 2
HPAD = 25600
CHUNK = 800
HCHUNKS = HPAD // CHUNK
RCHUNKS = HCHUNKS // 8
NPAD = 2 * HPAD


_MASK_HI = np.int32(-65536)


def _unpack_pair(g32):
    amp = plsc.bitcast(g32 & _MASK_HI, jnp.float32)
    ph = plsc.bitcast(g32 << 16, jnp.float32)
    return amp, ph


def _sc_smooth_body(half_base, params_hbm, idx_hbm, w_hbm, out_hbm,
                    table_v, idx_v, w_v, oa0, oa1, op0, op1,
                    sem_idx, sem_w, sem_oa, sem_op):
    oa = (oa0, oa1)
    op = (op0, op1)
    cid = lax.axis_index("c")
    sid = lax.axis_index("s")
    r = sid % 4
    e = (sid // 4) * 2 + cid

    t_off = pl.multiple_of(r * NPAD, NPAD)
    pltpu.sync_copy(params_hbm.at[pl.ds(t_off, NPAD)], table_v)

    def in_copies(blk):
        buf = blk % 2
        g = e * RCHUNKS + blk
        return (
            pltpu.make_async_copy(idx_hbm.at[g], idx_v.at[buf],
                                  sem_idx.at[buf]),
            pltpu.make_async_copy(w_hbm.at[g], w_v.at[buf],
                                  sem_w.at[buf]),
        )

    def out_copies(blk):
        buf = blk % 2
        c_i = e * RCHUNKS + blk
        o_a = pl.multiple_of(r * HPAD + c_i * CHUNK, CHUNK)
        o_p = pl.multiple_of((r + 4) * HPAD + c_i * CHUNK, CHUNK)
        return (
            pltpu.make_async_copy(oa[buf], out_hbm.at[pl.ds(o_a, CHUNK)],
                                  sem_oa.at[buf]),
            pltpu.make_async_copy(op[buf], out_hbm.at[pl.ds(o_p, CHUNK)],
                                  sem_op.at[buf]),
        )

    for c in in_copies(0):
        c.start()
    for blk in range(RCHUNKS):
        buf = blk % 2
        if blk + 1 < RCHUNKS:
            for c in in_copies(blk + 1):
                c.start()
        for c in in_copies(blk):
            c.wait()
        if blk >= 2:
            for c in out_copies(blk - 2):
                c.wait()
        base = half_base + (e * RCHUNKS + blk) * CHUNK

        def body(gi, _, buf=buf, base=base):
            col = gi * 16
            g32 = plsc.load_gather(table_v, [idx_v[buf, 0, pl.ds(col, 16)]])
            amp, ph = _unpack_pair(g32)
            wv = w_v[buf, 0, pl.ds(col, 16)]
            acc_a = wv * amp
            acc_p = wv * ph
            for k in range(1, K):
                g32 = plsc.load_gather(table_v, [idx_v[buf, k, pl.ds(col, 16)]])
                amp, ph = _unpack_pair(g32)
                wv = w_v[buf, k, pl.ds(col, 16)]
                acc_a = acc_a + wv * amp
                acc_p = acc_p + wv * ph
            ca, cp = _unpack_pair(table_v[pl.ds(base + col, 16)])
            oa[buf][pl.ds(col, 16)] = (1.0 - SMOOTH) * ca + SMOOTH * acc_a
            op[buf][pl.ds(col, 16)] = (1.0 - SMOOTH) * cp + SMOOTH * acc_p
            return 0

        lax.fori_loop(0, CHUNK // 16, body, 0)
        for c in out_copies(blk):
            c.start()
    for blk in (RCHUNKS - 2, RCHUNKS - 1):
        for c in out_copies(blk):
            c.wait()


_sc_cache = {}


def _get_sc_smooth(half):
    if half not in _sc_cache:
        _sc_cache[half] = pl.kernel(
            functools.partial(_sc_smooth_body, half * NH),
            out_type=jax.ShapeDtypeStruct((NPARAM * HPAD,), jnp.float32),
            mesh=plsc.VectorSubcoreMesh(core_axis_name="c",
                                        subcore_axis_name="s"),
            scratch_types=[
                pltpu.VMEM((NPAD,), jnp.int32),
                pltpu.VMEM((2, K, CHUNK), jnp.int32),
                pltpu.VMEM((2, K, CHUNK), jnp.float32),
                pltpu.VMEM((CHUNK,), jnp.float32),
                pltpu.VMEM((CHUNK,), jnp.float32),
                pltpu.VMEM((CHUNK,), jnp.float32),
                pltpu.VMEM((CHUNK,), jnp.float32),
                pltpu.SemaphoreType.DMA((2,)),
                pltpu.SemaphoreType.DMA((2,)),
                pltpu.SemaphoreType.DMA((2,)),
                pltpu.SemaphoreType.DMA((2,)),
            ],
            compiler_params=pltpu.CompilerParams(needs_layout_passes=False),
        )
    return _sc_cache[half]


_HALF_PI = np.float32(np.pi / 2)
_QUARTER_PI = np.float32(np.pi / 4)
_TWO_OVER_PI = np.float32(2.0 / np.pi)
_INV_SQRT2 = np.float32(1.0 / np.sqrt(2.0))


def _fast_cos(x):
    u = x * _TWO_OVER_PI
    kf = jnp.floor(u)
    t = (u - kf) * _HALF_PI - _QUARTER_PI
    t2 = t * t
    st = t * (1.0 + t2 * (np.float32(-1.0 / 6.0) + t2 * (
        np.float32(1.0 / 120.0) + t2 * np.float32(-1.0 / 5040.0))))
    ct = 1.0 + t2 * (np.float32(-0.5) + t2 * (
        np.float32(1.0 / 24.0) + t2 * np.float32(-1.0 / 720.0)))
    s = (ct + st) * _INV_SQRT2
    c = (ct - st) * _INV_SQRT2
    ki = kf.astype(jnp.int32)
    mag = jnp.where((ki & 1) == 1, s, c)
    return jnp.where(((ki + 1) & 2) == 2, -mag, mag)


def _bf16_split(x):
    hi = x.astype(jnp.bfloat16).astype(jnp.float32)
    return hi, x - hi


def _coeff_body(time_ref, off_ref, trend_ref, sm_ref, basis_ref, co_ref):
    t = time_ref[...]
    args = []
    for period in PERIODS:
        w = np.float32(2.0 * np.pi / period)
        args.append(w * t - _HALF_PI)
    for period in PERIODS:
        w = np.float32(2.0 * np.pi / period)
        args.append(w * t)
    sincos_t = _fast_cos(jnp.concatenate(args, axis=0))
    t_hi, t_lo = _bf16_split(t)
    ones = jnp.ones_like(t)
    basis_ref[...] = jnp.concatenate(
        [sincos_t, ones, ones, t_hi, t_lo, t_hi,
         jnp.zeros((3, T), jnp.float32)], axis=0)

    amp = sm_ref[0:4, :]
    ph = sm_ref[4:8, :]
    sc = _fast_cos(jnp.concatenate([ph - _HALF_PI, ph], axis=0))
    off_hi, off_lo = _bf16_split(off_ref[...])
    tr_hi, tr_lo = _bf16_split(trend_ref[...])
    co = jnp.concatenate(
        [amp * sc[4:8, :],
         amp * sc[0:4, :],
         off_hi, off_lo, tr_hi, tr_hi, tr_lo,
         jnp.zeros((3, HPAD), jnp.float32)], axis=0)
    co_ref[...] = jnp.transpose(co)


def _coeffs(time2, off2, trend2, sm):
    return pl.pallas_call(
        _coeff_body,
        out_shape=(
            jax.ShapeDtypeStruct((16, T), jnp.float32),
            jax.ShapeDtypeStruct((HPAD, 16), jnp.float32),
        ),
    )(time2, off2, trend2, sm)


_ROWS = 5000


def _dense_body(basis_ref, co_ref, out_ref):
    out_ref[...] = jax.lax.dot_general(
        co_ref[...], basis_ref[...],
        dimension_numbers=(((1,), (0,)), ((), ())),
        preferred_element_type=jnp.float32)


def _dense_body_carry(basis_ref, co_ref, _, out_ref):
    _dense_body(basis_ref, co_ref, out_ref)


def _dense_half(half, basis, co, carry):
    nblk = NH // _ROWS
    in_specs = [
        pl.BlockSpec((16, T), lambda i: (0, 0)),
        pl.BlockSpec((_ROWS, 16), lambda i: (i, 0)),
    ]
    args = [basis, co]
    body = _dense_body
    aliases = {}
    if carry is not None:
        in_specs.append(pl.BlockSpec(memory_space=pl.ANY))
        args.append(carry)
        body = _dense_body_carry
        aliases = {2: 0}
    return pl.pallas_call(
        body,
        grid=(nblk,),
        in_specs=in_specs,
        out_specs=pl.BlockSpec((_ROWS, T),
                               lambda i, h=half: (h * nblk + i, 0)),
        out_shape=jax.ShapeDtypeStruct((N, T), jnp.float32),
        input_output_aliases=aliases,
    )(*args)


def kernel(time_vector, constant_offset, seasonal_amplitudes, seasonal_phases,
           linear_trend, neighbor_indices, neighbor_weights):
    amp_u = lax.bitcast_convert_type(
        seasonal_amplitudes.T.astype(jnp.bfloat16), jnp.uint16)
    ph_u = lax.bitcast_convert_type(
        seasonal_phases.T.astype(jnp.bfloat16), jnp.uint16)
    packed = (amp_u.astype(jnp.uint32) << 16) | ph_u.astype(jnp.uint32)
    packed = lax.bitcast_convert_type(packed, jnp.int32)
    packed = jnp.pad(packed, ((0, 0), (0, NPAD - N))).reshape(4 * NPAD)
    idxT = neighbor_indices.T.astype(jnp.int32)
    wT = neighbor_weights.T
    halves_idx = []
    halves_w = []
    for half in range(2):
        ih = jnp.pad(idxT[:, half * NH:(half + 1) * NH], ((0, 0), (0, HPAD - NH)))
        wh = jnp.pad(wT[:, half * NH:(half + 1) * NH], ((0, 0), (0, HPAD - NH)))
        halves_idx.append(ih.reshape(K, HCHUNKS, CHUNK).transpose(1, 0, 2))
        halves_w.append(wh.reshape(K, HCHUNKS, CHUNK).transpose(1, 0, 2))

    time2 = time_vector[None, :]
    sms = [_get_sc_smooth(half)(packed, halves_idx[half], halves_w[half])
           for half in range(2)]
    stage2 = []
    for half in range(2):
        sm = sms[half].reshape(NPARAM, HPAD)
        off_h = jnp.pad(constant_offset[half * NH:(half + 1) * NH],
                        (0, HPAD - NH))[None, :]
        tr_h = jnp.pad(linear_trend[half * NH:(half + 1) * NH],
                       (0, HPAD - NH))[None, :]
        stage2.append(_coeffs(time2, off_h, tr_h, sm))
    carry = None
    for half in range(2):
        basis, co = stage2[half]
        carry = _dense_half(half, basis, co, carry)
    return carry


# speedup vs baseline: 1.4760x; 1.0021x over previous
import functools

import jax
import jax.numpy as jnp
import numpy as np
from jax import lax
from jax.experimental import pallas as pl
from jax.experimental.pallas import tpu as pltpu
from jax.experimental.pallas import tpu_sc as plsc

N = 50000
K = 16
T = 512
NPARAM = 8
SMOOTH = 0.1
PERIODS = (0.25, 0.5, 1.0, 2.0)

NH = N // 2
HPAD = 25600
CHUNK = 800
HCHUNKS = HPAD // CHUNK
RCHUNKS = HCHUNKS // 8
NPAD = 2 * HPAD


_MASK_HI = np.int32(-65536)


def _unpack_pair(g32):
    amp = plsc.bitcast(g32 & _MASK_HI, jnp.float32)
    ph = plsc.bitcast(g32 << 16, jnp.float32)
    return amp, ph


def _sc_smooth_body(half_base, params_hbm, idx_hbm, w_hbm, out_hbm,
                    table_v, idx_v, w_v, oa0, oa1, op0, op1, shared_tab,
                    sem_idx, sem_w, sem_oa, sem_op):
    oa = (oa0, oa1)
    op = (op0, op1)
    cid = lax.axis_index("c")
    sid = lax.axis_index("s")
    r = sid % 4
    e = (sid // 4) * 2 + cid

    sid_off = pl.multiple_of(sid * NPAD, NPAD)

    @pl.when(sid < 4)
    def _stage_tables():
        pltpu.sync_copy(params_hbm.at[pl.ds(sid_off, NPAD)],
                        shared_tab.at[pl.ds(sid_off, NPAD)])

    plsc.subcore_barrier()
    t_off = pl.multiple_of(r * NPAD, NPAD)
    pltpu.sync_copy(shared_tab.at[pl.ds(t_off, NPAD)], table_v)

    def in_copies(blk):
        buf = blk % 2
        g = e * RCHUNKS + blk
        return (
            pltpu.make_async_copy(idx_hbm.at[g], idx_v.at[buf],
                                  sem_idx.at[buf]),
            pltpu.make_async_copy(w_hbm.at[g], w_v.at[buf],
                                  sem_w.at[buf]),
        )

    def out_copies(blk):
        buf = blk % 2
        c_i = e * RCHUNKS + blk
        o_a = pl.multiple_of(r * HPAD + c_i * CHUNK, CHUNK)
        o_p = pl.multiple_of((r + 4) * HPAD + c_i * CHUNK, CHUNK)
        return (
            pltpu.make_async_copy(oa[buf], out_hbm.at[pl.ds(o_a, CHUNK)],
                                  sem_oa.at[buf]),
            pltpu.make_async_copy(op[buf], out_hbm.at[pl.ds(o_p, CHUNK)],
                                  sem_op.at[buf]),
        )

    for c in in_copies(0):
        c.start()
    for blk in range(RCHUNKS):
        buf = blk % 2
        if blk + 1 < RCHUNKS:
            for c in in_copies(blk + 1):
                c.start()
        for c in in_copies(blk):
            c.wait()
        if blk >= 2:
            for c in out_copies(blk - 2):
                c.wait()
        base = half_base + (e * RCHUNKS + blk) * CHUNK

        def body(gi, _, buf=buf, base=base):
            col = gi * 16
            g32 = plsc.load_gather(table_v, [idx_v[buf, 0, pl.ds(col, 16)]])
            amp, ph = _unpack_pair(g32)
            wv = w_v[buf, 0, pl.ds(col, 16)]
            acc_a = wv * amp
            acc_p = wv * ph
            for k in range(1, K):
                g32 = plsc.load_gather(table_v, [idx_v[buf, k, pl.ds(col, 16)]])
                amp, ph = _unpack_pair(g32)
                wv = w_v[buf, k, pl.ds(col, 16)]
                acc_a = acc_a + wv * amp
                acc_p = acc_p + wv * ph
            ca, cp = _unpack_pair(table_v[pl.ds(base + col, 16)])
            oa[buf][pl.ds(col, 16)] = (1.0 - SMOOTH) * ca + SMOOTH * acc_a
            op[buf][pl.ds(col, 16)] = (1.0 - SMOOTH) * cp + SMOOTH * acc_p
            return 0

        lax.fori_loop(0, CHUNK // 16, body, 0)
        for c in out_copies(blk):
            c.start()
    for blk in (RCHUNKS - 2, RCHUNKS - 1):
        for c in out_copies(blk):
            c.wait()


_sc_cache = {}


def _get_sc_smooth(half):
    if half not in _sc_cache:
        _sc_cache[half] = pl.kernel(
            functools.partial(_sc_smooth_body, half * NH),
            out_type=jax.ShapeDtypeStruct((NPARAM * HPAD,), jnp.float32),
            mesh=plsc.VectorSubcoreMesh(core_axis_name="c",
                                        subcore_axis_name="s"),
            scratch_types=[
                pltpu.VMEM((NPAD,), jnp.int32),
                pltpu.VMEM((2, K, CHUNK), jnp.int32),
                pltpu.VMEM((2, K, CHUNK), jnp.float32),
                pltpu.VMEM((CHUNK,), jnp.float32),
                pltpu.VMEM((CHUNK,), jnp.float32),
                pltpu.VMEM((CHUNK,), jnp.float32),
                pltpu.VMEM((CHUNK,), jnp.float32),
                pltpu.VMEM_SHARED((4 * NPAD,), jnp.int32),
                pltpu.SemaphoreType.DMA((2,)),
                pltpu.SemaphoreType.DMA((2,)),
                pltpu.SemaphoreType.DMA((2,)),
                pltpu.SemaphoreType.DMA((2,)),
            ],
            compiler_params=pltpu.CompilerParams(needs_layout_passes=False),
        )
    return _sc_cache[half]


_HALF_PI = np.float32(np.pi / 2)
_QUARTER_PI = np.float32(np.pi / 4)
_TWO_OVER_PI = np.float32(2.0 / np.pi)
_INV_SQRT2 = np.float32(1.0 / np.sqrt(2.0))


def _fast_cos(x):
    u = x * _TWO_OVER_PI
    kf = jnp.floor(u)
    t = (u - kf) * _HALF_PI - _QUARTER_PI
    t2 = t * t
    st = t * (1.0 + t2 * (np.float32(-1.0 / 6.0) + t2 * (
        np.float32(1.0 / 120.0) + t2 * np.float32(-1.0 / 5040.0))))
    ct = 1.0 + t2 * (np.float32(-0.5) + t2 * (
        np.float32(1.0 / 24.0) + t2 * np.float32(-1.0 / 720.0)))
    s = (ct + st) * _INV_SQRT2
    c = (ct - st) * _INV_SQRT2
    ki = kf.astype(jnp.int32)
    mag = jnp.where((ki & 1) == 1, s, c)
    return jnp.where(((ki + 1) & 2) == 2, -mag, mag)


def _bf16_split(x):
    hi = x.astype(jnp.bfloat16).astype(jnp.float32)
    return hi, x - hi


def _coeff_body(time_ref, off_ref, trend_ref, sm_ref, basis_ref, co_ref):
    t = time_ref[...]
    args = []
    for period in PERIODS:
        w = np.float32(2.0 * np.pi / period)
        args.append(w * t - _HALF_PI)
    for period in PERIODS:
        w = np.float32(2.0 * np.pi / period)
        args.append(w * t)
    sincos_t = _fast_cos(jnp.concatenate(args, axis=0))
    t_hi, t_lo = _bf16_split(t)
    ones = jnp.ones_like(t)
    basis_ref[...] = jnp.concatenate(
        [sincos_t, ones, ones, t_hi, t_lo, t_hi,
         jnp.zeros((3, T), jnp.float32)], axis=0)

    amp = sm_ref[0:4, :]
    ph = sm_ref[4:8, :]
    sc = _fast_cos(jnp.concatenate([ph - _HALF_PI, ph], axis=0))
    off_hi, off_lo = _bf16_split(off_ref[...])
    tr_hi, tr_lo = _bf16_split(trend_ref[...])
    co = jnp.concatenate(
        [amp * sc[4:8, :],
         amp * sc[0:4, :],
         off_hi, off_lo, tr_hi, tr_hi, tr_lo,
         jnp.zeros((3, HPAD), jnp.float32)], axis=0)
    co_ref[...] = jnp.transpose(co)


def _coeffs(time2, off2, trend2, sm):
    return pl.pallas_call(
        _coeff_body,
        out_shape=(
            jax.ShapeDtypeStruct((16, T), jnp.float32),
            jax.ShapeDtypeStruct((HPAD, 16), jnp.float32),
        ),
    )(time2, off2, trend2, sm)


_ROWS = 5000


def _dense_body(basis_ref, co_ref, out_ref):
    out_ref[...] = jax.lax.dot_general(
        co_ref[...], basis_ref[...],
        dimension_numbers=(((1,), (0,)), ((), ())),
        preferred_element_type=jnp.float32)


def _dense_body_carry(basis_ref, co_ref, _, out_ref):
    _dense_body(basis_ref, co_ref, out_ref)


def _dense_half(half, basis, co, carry):
    nblk = NH // _ROWS
    in_specs = [
        pl.BlockSpec((16, T), lambda i: (0, 0)),
        pl.BlockSpec((_ROWS, 16), lambda i: (i, 0)),
    ]
    args = [basis, co]
    body = _dense_body
    aliases = {}
    if carry is not None:
        in_specs.append(pl.BlockSpec(memory_space=pl.ANY))
        args.append(carry)
        body = _dense_body_carry
        aliases = {2: 0}
    return pl.pallas_call(
        body,
        grid=(nblk,),
        in_specs=in_specs,
        out_specs=pl.BlockSpec((_ROWS, T),
                               lambda i, h=half: (h * nblk + i, 0)),
        out_shape=jax.ShapeDtypeStruct((N, T), jnp.float32),
        input_output_aliases=aliases,
    )(*args)


def kernel(time_vector, constant_offset, seasonal_amplitudes, seasonal_phases,
           linear_trend, neighbor_indices, neighbor_weights):
    amp_u = lax.bitcast_convert_type(
        seasonal_amplitudes.T.astype(jnp.bfloat16), jnp.uint16)
    ph_u = lax.bitcast_convert_type(
        seasonal_phases.T.astype(jnp.bfloat16), jnp.uint16)
    packed = (amp_u.astype(jnp.uint32) << 16) | ph_u.astype(jnp.uint32)
    packed = lax.bitcast_convert_type(packed, jnp.int32)
    packed = jnp.pad(packed, ((0, 0), (0, NPAD - N))).reshape(4 * NPAD)
    idxT = neighbor_indices.T.astype(jnp.int32)
    wT = neighbor_weights.T
    halves_idx = []
    halves_w = []
    for half in range(2):
        ih = jnp.pad(idxT[:, half * NH:(half + 1) * NH], ((0, 0), (0, HPAD - NH)))
        wh = jnp.pad(wT[:, half * NH:(half + 1) * NH], ((0, 0), (0, HPAD - NH)))
        halves_idx.append(ih.reshape(K, HCHUNKS, CHUNK).transpose(1, 0, 2))
        halves_w.append(wh.reshape(K, HCHUNKS, CHUNK).transpose(1, 0, 2))

    time2 = time_vector[None, :]
    sms = [_get_sc_smooth(half)(packed, halves_idx[half], halves_w[half])
           for half in range(2)]
    stage2 = []
    for half in range(2):
        sm = sms[half].reshape(NPARAM, HPAD)
        off_h = jnp.pad(constant_offset[half * NH:(half + 1) * NH],
                        (0, HPAD - NH))[None, :]
        tr_h = jnp.pad(linear_trend[half * NH:(half + 1) * NH],
                       (0, HPAD - NH))[None, :]
        stage2.append(_coeffs(time2, off_h, tr_h, sm))
    carry = None
    for half in range(2):
        basis, co = stage2[half]
        carry = _dense_half(half, basis, co, carry)
    return carry
